# baseline (speedup 1.0000x reference)
import functools

import jax
import jax.numpy as jnp
from jax import lax
from jax.experimental import pallas as pl
from jax.experimental.pallas import tpu as pltpu
from jax.experimental.pallas import tpu_sc as plsc

_BATCH = 16384
_CHUNK = 128
_TABLE = 1000


@functools.cache
def _build_kernel():
    info = plsc.get_sparse_core_info()
    num_cores, num_subcores = info.num_cores, info.num_subcores
    num_workers = num_cores * num_subcores
    b_per_w = _BATCH // num_workers
    n_chunks = b_per_w // _CHUNK

    mesh = plsc.VectorSubcoreMesh(core_axis_name="c", subcore_axis_name="s")

    @functools.partial(
        pl.kernel,
        mesh=mesh,
        out_type=(
            jax.ShapeDtypeStruct((num_workers, n_chunks, _CHUNK), jnp.float32),
            jax.ShapeDtypeStruct((num_workers, n_chunks, _CHUNK), jnp.float32),
        ),
        scratch_types=[
            pltpu.VMEM((n_chunks, _CHUNK), jnp.int32),
            pltpu.VMEM((n_chunks, _CHUNK), jnp.float32),
            pltpu.VMEM((n_chunks, _CHUNK), jnp.float32),
            pltpu.VMEM_SHARED((_TABLE,), jnp.float32),
            pltpu.VMEM_SHARED((_TABLE,), jnp.float32),
            pltpu.SemaphoreType.DMA,
            pltpu.SemaphoreType.DMA,
            pltpu.SemaphoreType.DMA,
        ],
    )
    def ddpm_lookup(
        t_hbm,
        beta_hbm,
        alpha_hbm,
        beta_out_hbm,
        alpha_out_hbm,
        idx_v,
        beta_o_v,
        alpha_o_v,
        beta_s,
        alpha_s,
        gsem,
        asem,
        osem,
    ):
        sid = lax.axis_index("s")
        wid = sid * num_cores + lax.axis_index("c")

        @pl.when(sid == 0)
        def _stage_beta():
            pltpu.sync_copy(beta_hbm, beta_s)

        @pl.when(sid == 1)
        def _stage_alpha():
            pltpu.sync_copy(alpha_hbm, alpha_s)

        pltpu.sync_copy(t_hbm.at[wid], idx_v)
        plsc.subcore_barrier()
        beta_gathers = [
            pltpu.async_copy(beta_s.at[idx_v.at[j]], beta_o_v.at[j], gsem)
            for j in range(n_chunks)
        ]
        alpha_gathers = [
            pltpu.async_copy(alpha_s.at[idx_v.at[j]], alpha_o_v.at[j], asem)
            for j in range(n_chunks)
        ]
        for d in beta_gathers:
            d.wait()
        o1 = pltpu.async_copy(beta_o_v, beta_out_hbm.at[wid], osem)
        for d in alpha_gathers:
            d.wait()
        o2 = pltpu.async_copy(alpha_o_v, alpha_out_hbm.at[wid], osem)
        o1.wait()
        o2.wait()

    return ddpm_lookup, num_workers, n_chunks


def kernel(t, beta, alpha):
    fn, num_workers, n_chunks = _build_kernel()
    t3 = t.astype(jnp.int32).reshape(num_workers, n_chunks, _CHUNK)
    beta_t, alpha_t = fn(t3, beta, alpha)
    return beta_t.reshape(_BATCH), alpha_t.reshape(_BATCH)


# speedup vs baseline: 1.0042x; 1.0042x over previous
import functools

import jax
import jax.numpy as jnp
from jax import lax
from jax.experimental import pallas as pl
from jax.experimental.pallas import tpu as pltpu
from jax.experimental.pallas import tpu_sc as plsc

_BATCH = 16384
_CHUNK = 128
_TABLE = 1000


@functools.cache
def _build_kernel():
    info = plsc.get_sparse_core_info()
    num_cores, num_subcores = info.num_cores, info.num_subcores
    num_workers = num_cores * num_subcores
    b_per_w = _BATCH // num_workers
    n_chunks = b_per_w // _CHUNK

    mesh = plsc.VectorSubcoreMesh(core_axis_name="c", subcore_axis_name="s")

    @functools.partial(
        pl.kernel,
        mesh=mesh,
        out_type=(
            jax.ShapeDtypeStruct((num_workers, n_chunks, _CHUNK), jnp.float32),
            jax.ShapeDtypeStruct((num_workers, n_chunks, _CHUNK), jnp.float32),
        ),
        scratch_types=[
            pltpu.VMEM((n_chunks, _CHUNK), jnp.int32),
            pltpu.VMEM((n_chunks, _CHUNK), jnp.float32),
            pltpu.VMEM((n_chunks, _CHUNK), jnp.float32),
            pltpu.VMEM_SHARED((_TABLE,), jnp.float32),
            pltpu.VMEM_SHARED((_TABLE,), jnp.float32),
            pltpu.SemaphoreType.DMA,
            pltpu.SemaphoreType.DMA,
            pltpu.SemaphoreType.DMA,
            pltpu.SemaphoreType.DMA,
        ],
    )
    def ddpm_lookup(
        t_hbm,
        beta_hbm,
        alpha_hbm,
        beta_out_hbm,
        alpha_out_hbm,
        idx_v,
        beta_o_v,
        alpha_o_v,
        beta_s,
        alpha_s,
        gsem,
        asem,
        osem,
        isem,
    ):
        sid = lax.axis_index("s")
        wid = sid * num_cores + lax.axis_index("c")

        @pl.when(sid == 0)
        def _stage_beta():
            pltpu.sync_copy(beta_hbm, beta_s)

        @pl.when(sid == 1)
        def _stage_alpha():
            pltpu.sync_copy(alpha_hbm, alpha_s)

        idx_copies = [
            pltpu.async_copy(t_hbm.at[wid, j], idx_v.at[j], isem)
            for j in range(n_chunks)
        ]
        plsc.subcore_barrier()
        beta_gathers = []
        alpha_gathers = []
        for j in range(n_chunks):
            idx_copies[j].wait()
            beta_gathers.append(
                pltpu.async_copy(beta_s.at[idx_v.at[j]], beta_o_v.at[j], gsem)
            )
            alpha_gathers.append(
                pltpu.async_copy(alpha_s.at[idx_v.at[j]], alpha_o_v.at[j], asem)
            )
        outs = []
        for j in range(n_chunks):
            beta_gathers[j].wait()
            outs.append(
                pltpu.async_copy(beta_o_v.at[j], beta_out_hbm.at[wid, j], osem)
            )
            alpha_gathers[j].wait()
            outs.append(
                pltpu.async_copy(alpha_o_v.at[j], alpha_out_hbm.at[wid, j], osem)
            )
        for d in outs:
            d.wait()

    return ddpm_lookup, num_workers, n_chunks


def kernel(t, beta, alpha):
    fn, num_workers, n_chunks = _build_kernel()
    t3 = t.astype(jnp.int32).reshape(num_workers, n_chunks, _CHUNK)
    beta_t, alpha_t = fn(t3, beta, alpha)
    return beta_t.reshape(_BATCH), alpha_t.reshape(_BATCH)
